# baseline (speedup 1.0000x reference)
import functools

import jax
import jax.numpy as jnp
from jax import lax
from jax.experimental import pallas as pl
from jax.experimental.pallas import tpu as pltpu
from jax.experimental.pallas import tpu_sc as plsc

_V = 1000000
_D = 64
_IMG = 2048
_B = 16384
_NNEG = 20

_NW = 32
_BPW = _B // _NW
_C = 128
_NSB = _BPW // _C


_NR = 4
_NIPC = _NR * _NNEG
_NCH = _BPW // _NR
_NBUF = 2


_PD = 2 * _D


def _splat(vec16, lane):
    return lax.gather(
        vec16, jnp.full((16, 1), lane, jnp.int32),
        lax.GatherDimensionNumbers(offset_dims=(), collapsed_slice_dims=(0,),
                                   start_index_map=(0,)),
        slice_sizes=(1,), mode=lax.GatherScatterMode.PROMISE_IN_BOUNDS)


def _sc_gather_body(u_hbm, v_hbm, vneg_hbm, U_hbm, V_hbm,
                    eu_out, ev_out, evs_out,
                    uidx, upo, vidx, vpo, nidx, npo,
                    half, acc, pbuf, nbuf,
                    sem_uv, sem_n):
    wid = lax.axis_index("s") * 2 + lax.axis_index("c")
    base = wid * _BPW
    pltpu.sync_copy(u_hbm.at[pl.ds(base, _BPW)], uidx)
    pltpu.sync_copy(v_hbm.at[pl.ds(base, _BPW)], vidx)
    pltpu.sync_copy(vneg_hbm.at[pl.ds(base * _NNEG, _BPW * _NNEG)], nidx)

    iota = lax.iota(jnp.int32, 16)
    cvec = [iota + c4 * 16 for c4 in range(_D // 16)]

    def xform(i, carry):
        sl = pl.ds(i * 16, 16)
        for idx_ref, po_ref in ((uidx, upo), (vidx, vpo)):
            t = idx_ref[sl]
            idx_ref[sl] = lax.shift_right_logical(t, 1)
            po_ref[sl] = lax.shift_left((t & 1), 6)
        return carry

    lax.fori_loop(0, _BPW // 16, xform, 0)

    def nxform(i, carry):
        sl = pl.ds(i * 16, 16)
        t = nidx[sl]
        nidx[sl] = lax.shift_right_logical(t, 1)
        npo[sl] = lax.shift_left((t & 1), 6)
        return carry

    lax.fori_loop(0, (_BPW * _NNEG) // 16, nxform, 0)

    jobs = []
    for k in range(_NSB):
        jobs.append((U_hbm, uidx, upo, eu_out, k))
        jobs.append((V_hbm, vidx, vpo, ev_out, k))

    def fire_uv(t):
        tbl, idx, _po, _out, k = jobs[t]
        return pltpu.async_copy(
            tbl.at[idx.at[pl.ds(k * _C, _C)]], pbuf.at[t % 2], sem_uv)

    cps = [fire_uv(0), fire_uv(1)]
    for t in range(2 * _NSB):
        cps[t].wait()
        tbl, idx, po, out, k = jobs[t]

        def ebody(g, carry, po=po, t=t):
            pv = po[pl.ds(k * _C + g * 16, 16)]
            for lane in range(16):
                spl = _splat(pv, lane)
                row = g * 16 + lane
                rowv = jnp.full((16,), row, jnp.int32)
                for c4 in range(_D // 16):
                    half[row, pl.ds(c4 * 16, 16)] = plsc.load_gather(
                        pbuf, [jnp.full((16,), t % 2, jnp.int32), rowv,
                               spl + cvec[c4]])
            return carry

        lax.fori_loop(0, _C // 16, ebody, 0)
        if t + 2 < 2 * _NSB:
            cps.append(fire_uv(t + 2))
        pltpu.sync_copy(half, out.at[pl.ds(base + k * _C, _C)])

    def fire_n(c, buf):
        return pltpu.async_copy(
            V_hbm.at[nidx.at[pl.ds(c * _NIPC, _NIPC)]], nbuf.at[buf], sem_n)

    for b in range(_NBUF):
        fire_n(b, b)

    def nbody(i, carry):
        for par in range(_NBUF):
            c = i * _NBUF + par
            pltpu.make_async_copy(
                V_hbm.at[nidx.at[pl.ds(0, _NIPC)]], nbuf.at[par], sem_n).wait()
            pvs = [npo[pl.ds(c * _NIPC + s * 16, 16)] for s in range(_NIPC // 16)]
            bufv = jnp.full((16,), par, jnp.int32)
            for rr in range(_NR):
                xs = [None] * (_D // 16)
                for n in range(_NNEG):
                    j = rr * _NNEG + n
                    spl = _splat(pvs[j // 16], j % 16)
                    jv = jnp.full((16,), j, jnp.int32)
                    for c4 in range(_D // 16):
                        val = plsc.load_gather(nbuf, [bufv, jv, spl + cvec[c4]])
                        xs[c4] = val if n == 0 else xs[c4] + val
                for c4 in range(_D // 16):
                    acc[c * _NR + rr, pl.ds(c4 * 16, 16)] = xs[c4]

            @pl.when(c + _NBUF < _NCH)
            def _():
                fire_n(c + _NBUF, par)
        return carry

    lax.fori_loop(0, _NCH // _NBUF, nbody, 0)

    pltpu.sync_copy(acc, evs_out.at[pl.ds(base, _BPW)])


@functools.cache
def _make_sc_gather():
    return pl.kernel(
        _sc_gather_body,
        out_type=(
            jax.ShapeDtypeStruct((_B, _D), jnp.float32),
            jax.ShapeDtypeStruct((_B, _D), jnp.float32),
            jax.ShapeDtypeStruct((_B, _D), jnp.float32),
        ),
        mesh=plsc.VectorSubcoreMesh(core_axis_name="c", subcore_axis_name="s"),
        compiler_params=pltpu.CompilerParams(use_tc_tiling_on_sc=False,
                                             needs_layout_passes=False),
        scratch_types=[
            pltpu.VMEM((_BPW,), jnp.int32),
            pltpu.VMEM((_BPW,), jnp.int32),
            pltpu.VMEM((_BPW,), jnp.int32),
            pltpu.VMEM((_BPW,), jnp.int32),
            pltpu.VMEM((_BPW * _NNEG,), jnp.int32),
            pltpu.VMEM((_BPW * _NNEG,), jnp.int32),
            pltpu.VMEM((_C, _D), jnp.float32),
            pltpu.VMEM((_BPW, _D), jnp.float32),
            pltpu.VMEM((2, _C, _PD), jnp.float32),
            pltpu.VMEM((_NBUF, _NIPC, _PD), jnp.float32),
            pltpu.SemaphoreType.DMA,
            pltpu.SemaphoreType.DMA,
        ],
    )

_VBK = 512
_LBK = 2048


def _vis_body(vp_ref, w_ref, b_ref, out_ref):
    y = lax.dot_general(vp_ref[...], w_ref[...], (((1,), (1,)), ((), ())),
                        preferred_element_type=jnp.float32) + b_ref[...]
    n = jnp.sqrt(jnp.sum(y * y, axis=1, keepdims=True))
    out_ref[...] = y / n


_vis_call = pl.pallas_call(
    _vis_body,
    grid=(_B // _VBK,),
    in_specs=[
        pl.BlockSpec((_VBK, _IMG), lambda i: (i, 0)),
        pl.BlockSpec((_D, _IMG), lambda i: (0, 0)),
        pl.BlockSpec((1, _D), lambda i: (0, 0)),
    ],
    out_specs=pl.BlockSpec((_VBK, _D), lambda i: (i, 0)),
    out_shape=jax.ShapeDtypeStruct((_B, _D), jnp.float32),
)


def _logsig(x):
    return jnp.minimum(x, 0.0) - jnp.log(1.0 + jnp.exp(-jnp.abs(x)))


def _loss_body(eu_ref, ev_ref, evs_ref, vis_ref, gw_ref, gb_ref, out_ref):
    i = pl.program_id(0)
    eu = eu_ref[...]
    z = lax.dot_general(eu, gw_ref[...], (((1,), (1,)), ((), ())),
                        preferred_element_type=jnp.float32) + gb_ref[...]
    gate = 1.0 / (1.0 + jnp.exp(-z))
    joint = eu + gate * vis_ref[...]
    score = jnp.sum(joint * ev_ref[...], axis=1)
    negsum = jnp.sum(joint * evs_ref[...], axis=1)
    part = jnp.sum(_logsig(score) + _logsig(-negsum))

    @pl.when(i == 0)
    def _():
        out_ref[...] = jnp.zeros((1, 1), jnp.float32)

    out_ref[...] += part


_loss_call = pl.pallas_call(
    _loss_body,
    grid=(_B // _LBK,),
    in_specs=[
        pl.BlockSpec((_LBK, _D), lambda i: (i, 0)),
        pl.BlockSpec((_LBK, _D), lambda i: (i, 0)),
        pl.BlockSpec((_LBK, _D), lambda i: (i, 0)),
        pl.BlockSpec((_LBK, _D), lambda i: (i, 0)),
        pl.BlockSpec((_D, _D), lambda i: (0, 0)),
        pl.BlockSpec((1, _D), lambda i: (0, 0)),
    ],
    out_specs=pl.BlockSpec((1, 1), lambda i: (0, 0)),
    out_shape=jax.ShapeDtypeStruct((1, 1), jnp.float32),
)


def kernel(u_pos, v_pos, v_neg, visual_pos, batch_size,
           U_emb, V_emb, gate_W, gate_b, img_W, img_b):
    u = u_pos.astype(jnp.int32)
    v = v_pos.astype(jnp.int32)
    vn = v_neg.astype(jnp.int32).reshape(-1)
    U2 = U_emb.reshape(_V // 2, _PD)
    V2 = V_emb.reshape(_V // 2, _PD)
    eu, ev, evs = _make_sc_gather()(u, v, vn, U2, V2)
    vis = _vis_call(visual_pos, img_W, img_b.reshape(1, _D))
    res = _loss_call(eu, ev, evs, vis, gate_W, gate_b.reshape(1, _D))
    return -res[0, 0] / batch_size


# speedup vs baseline: 1.1724x; 1.1724x over previous
import functools

import jax
import jax.numpy as jnp
from jax import lax
from jax.experimental import pallas as pl
from jax.experimental.pallas import tpu as pltpu
from jax.experimental.pallas import tpu_sc as plsc

_V = 1000000
_D = 64
_IMG = 2048
_B = 16384
_NNEG = 20

_NW = 32
_BPW = _B // _NW
_C = 128
_NSB = _BPW // _C


_NR = 4
_NIPC = _NR * _NNEG
_NCH = _BPW // _NR
_NBUF = 2


_PD = 2 * _D


def _sc_gather_body(u_hbm, v_hbm, vneg_hbm, T_hbm,
                    eu_out, ev_out, evs_out,
                    uidx, vidx, nidx, acc, pbuf, nbuf,
                    sem_uv, sem_n):
    wid = lax.axis_index("s") * 2 + lax.axis_index("c")
    base = wid * _BPW
    pltpu.sync_copy(u_hbm.at[pl.ds(base, _BPW)], uidx)
    pltpu.sync_copy(v_hbm.at[pl.ds(base, _BPW)], vidx)
    pltpu.sync_copy(vneg_hbm.at[pl.ds(base * _NNEG, _BPW * _NNEG)], nidx)

    jobs = []
    for k in range(_NSB):
        jobs.append((uidx, 0, eu_out, k))
        jobs.append((vidx, _D, ev_out, k))

    def fire_uv(t):
        idx, _col, _out, k = jobs[t]
        return pltpu.async_copy(
            T_hbm.at[idx.at[pl.ds(k * _C, _C)]], pbuf.at[t % 2], sem_uv)

    cps = [fire_uv(0), fire_uv(1)]
    for t in range(2 * _NSB):
        cps[t].wait()
        _idx, col, out, k = jobs[t]
        if t + 2 < 2 * _NSB:
            cps.append(fire_uv(t + 2))
        pltpu.sync_copy(pbuf.at[t % 2, :, pl.ds(col, _D)],
                        out.at[pl.ds(base + k * _C, _C)])

    def fire_n(c, buf):
        return pltpu.async_copy(
            T_hbm.at[nidx.at[pl.ds(c * _NIPC, _NIPC)]], nbuf.at[buf], sem_n)

    for b in range(_NBUF):
        fire_n(b, b)

    def nbody(i, carry):
        for par in range(_NBUF):
            c = i * _NBUF + par
            pltpu.make_async_copy(
                T_hbm.at[nidx.at[pl.ds(0, _NIPC)]], nbuf.at[par], sem_n).wait()
            for rr in range(_NR):
                for c4 in range(_D // 16):
                    sl = pl.ds(_D + c4 * 16, 16)
                    x = nbuf[par, rr * _NNEG, sl]
                    for n in range(1, _NNEG):
                        x = x + nbuf[par, rr * _NNEG + n, sl]
                    acc[c * _NR + rr, pl.ds(c4 * 16, 16)] = x

            @pl.when(c + _NBUF < _NCH)
            def _():
                fire_n(c + _NBUF, par)
        return carry

    lax.fori_loop(0, _NCH // _NBUF, nbody, 0)

    pltpu.sync_copy(acc, evs_out.at[pl.ds(base, _BPW)])


@functools.cache
def _make_sc_gather():
    return pl.kernel(
        _sc_gather_body,
        out_type=(
            jax.ShapeDtypeStruct((_B, _D), jnp.float32),
            jax.ShapeDtypeStruct((_B, _D), jnp.float32),
            jax.ShapeDtypeStruct((_B, _D), jnp.float32),
        ),
        mesh=plsc.VectorSubcoreMesh(core_axis_name="c", subcore_axis_name="s"),
        compiler_params=pltpu.CompilerParams(use_tc_tiling_on_sc=False,
                                             needs_layout_passes=False),
        scratch_types=[
            pltpu.VMEM((_BPW,), jnp.int32),
            pltpu.VMEM((_BPW,), jnp.int32),
            pltpu.VMEM((_BPW * _NNEG,), jnp.int32),
            pltpu.VMEM((_BPW, _D), jnp.float32),
            pltpu.VMEM((2, _C, _PD), jnp.float32),
            pltpu.VMEM((_NBUF, _NIPC, _PD), jnp.float32),
            pltpu.SemaphoreType.DMA,
            pltpu.SemaphoreType.DMA,
        ],
    )

_VBK = 512
_LBK = 2048


def _vis_body(vp_ref, w_ref, b_ref, out_ref):
    y = lax.dot_general(vp_ref[...], w_ref[...], (((1,), (1,)), ((), ())),
                        preferred_element_type=jnp.float32) + b_ref[...]
    n = jnp.sqrt(jnp.sum(y * y, axis=1, keepdims=True))
    out_ref[...] = y / n


_vis_call = pl.pallas_call(
    _vis_body,
    grid=(_B // _VBK,),
    in_specs=[
        pl.BlockSpec((_VBK, _IMG), lambda i: (i, 0)),
        pl.BlockSpec((_D, _IMG), lambda i: (0, 0)),
        pl.BlockSpec((1, _D), lambda i: (0, 0)),
    ],
    out_specs=pl.BlockSpec((_VBK, _D), lambda i: (i, 0)),
    out_shape=jax.ShapeDtypeStruct((_B, _D), jnp.float32),
)


def _logsig(x):
    return jnp.minimum(x, 0.0) - jnp.log(1.0 + jnp.exp(-jnp.abs(x)))


def _loss_body(eu_ref, ev_ref, evs_ref, vis_ref, gw_ref, gb_ref, out_ref):
    i = pl.program_id(0)
    eu = eu_ref[...]
    z = lax.dot_general(eu, gw_ref[...], (((1,), (1,)), ((), ())),
                        preferred_element_type=jnp.float32) + gb_ref[...]
    gate = 1.0 / (1.0 + jnp.exp(-z))
    joint = eu + gate * vis_ref[...]
    score = jnp.sum(joint * ev_ref[...], axis=1)
    negsum = jnp.sum(joint * evs_ref[...], axis=1)
    part = jnp.sum(_logsig(score) + _logsig(-negsum))

    @pl.when(i == 0)
    def _():
        out_ref[...] = jnp.zeros((1, 1), jnp.float32)

    out_ref[...] += part


_loss_call = pl.pallas_call(
    _loss_body,
    grid=(_B // _LBK,),
    in_specs=[
        pl.BlockSpec((_LBK, _D), lambda i: (i, 0)),
        pl.BlockSpec((_LBK, _D), lambda i: (i, 0)),
        pl.BlockSpec((_LBK, _D), lambda i: (i, 0)),
        pl.BlockSpec((_LBK, _D), lambda i: (i, 0)),
        pl.BlockSpec((_D, _D), lambda i: (0, 0)),
        pl.BlockSpec((1, _D), lambda i: (0, 0)),
    ],
    out_specs=pl.BlockSpec((1, 1), lambda i: (0, 0)),
    out_shape=jax.ShapeDtypeStruct((1, 1), jnp.float32),
)


def kernel(u_pos, v_pos, v_neg, visual_pos, batch_size,
           U_emb, V_emb, gate_W, gate_b, img_W, img_b):
    u = u_pos.astype(jnp.int32)
    v = v_pos.astype(jnp.int32)
    vn = v_neg.astype(jnp.int32).reshape(-1)
    T = jnp.concatenate([U_emb, V_emb], axis=1)
    eu, ev, evs = _make_sc_gather()(u, v, vn, T)
    vis = _vis_call(visual_pos, img_W, img_b.reshape(1, _D))
    res = _loss_call(eu, ev, evs, vis, gate_W, gate_b.reshape(1, _D))
    return -res[0, 0] / batch_size


# speedup vs baseline: 1.3307x; 1.1350x over previous
import functools

import jax
import jax.numpy as jnp
from jax import lax
from jax.experimental import pallas as pl
from jax.experimental.pallas import tpu as pltpu
from jax.experimental.pallas import tpu_sc as plsc

_V = 1000000
_D = 64
_IMG = 2048
_B = 16384
_NNEG = 20

_NW = 32
_BPW = _B // _NW
_C = 128
_NSB = _BPW // _C


_NR = 4
_NIPC = _NR * _NNEG
_NCH = _BPW // _NR
_NBUF = 2


_PD = 2 * _D


def _sc_gather_body(u_hbm, v_hbm, vneg_hbm, T_hbm,
                    eu_out, ev_out, evs_out,
                    uidx, vidx, nidx, acc, pbuf, nbuf,
                    sem_uv, sem_n):
    wid = lax.axis_index("s") * 2 + lax.axis_index("c")
    base = wid * _BPW
    pltpu.sync_copy(u_hbm.at[pl.ds(base, _BPW)], uidx)
    pltpu.sync_copy(v_hbm.at[pl.ds(base, _BPW)], vidx)
    pltpu.sync_copy(vneg_hbm.at[pl.ds(base * _NNEG, _BPW * _NNEG)], nidx)

    jobs = []
    for k in range(_NSB):
        jobs.append((uidx, 0, eu_out, k))
        jobs.append((vidx, _D, ev_out, k))

    def fire_uv(t):
        idx, _col, _out, k = jobs[t]
        return pltpu.async_copy(
            T_hbm.at[idx.at[pl.ds(k * _C, _C)]], pbuf.at[t % 2], sem_uv)

    cps = [fire_uv(0), fire_uv(1)]
    for t in range(2 * _NSB):
        cps[t].wait()
        _idx, col, out, k = jobs[t]
        if t + 2 < 2 * _NSB:
            cps.append(fire_uv(t + 2))
        pltpu.sync_copy(pbuf.at[t % 2, :, pl.ds(col, _D)],
                        out.at[pl.ds(base + k * _C, _C)])

    def fire_n(c, buf):
        return pltpu.async_copy(
            T_hbm.at[nidx.at[pl.ds(c * _NIPC, _NIPC)]], nbuf.at[buf], sem_n)

    for b in range(_NBUF):
        fire_n(b, b)

    def nbody(i, carry):
        for par in range(_NBUF):
            c = i * _NBUF + par
            pltpu.make_async_copy(
                T_hbm.at[nidx.at[pl.ds(0, _NIPC)]], nbuf.at[par], sem_n).wait()
            for rr in range(_NR):
                for c4 in range(_D // 16):
                    sl = pl.ds(_D + c4 * 16, 16)
                    x = nbuf[par, rr * _NNEG, sl]
                    for n in range(1, _NNEG):
                        x = x + nbuf[par, rr * _NNEG + n, sl]
                    acc[c * _NR + rr, pl.ds(c4 * 16, 16)] = x

            @pl.when(c + _NBUF < _NCH)
            def _():
                fire_n(c + _NBUF, par)
        return carry

    lax.fori_loop(0, _NCH // _NBUF, nbody, 0)

    pltpu.sync_copy(acc, evs_out.at[pl.ds(base, _BPW)])


@functools.cache
def _make_sc_gather():
    return pl.kernel(
        _sc_gather_body,
        out_type=(
            jax.ShapeDtypeStruct((_B, _D), jnp.float32),
            jax.ShapeDtypeStruct((_B, _D), jnp.float32),
            jax.ShapeDtypeStruct((_B, _D), jnp.float32),
        ),
        mesh=plsc.VectorSubcoreMesh(core_axis_name="c", subcore_axis_name="s"),
        compiler_params=pltpu.CompilerParams(use_tc_tiling_on_sc=False,
                                             needs_layout_passes=False),
        scratch_types=[
            pltpu.VMEM((_BPW,), jnp.int32),
            pltpu.VMEM((_BPW,), jnp.int32),
            pltpu.VMEM((_BPW * _NNEG,), jnp.int32),
            pltpu.VMEM((_BPW, _D), jnp.float32),
            pltpu.VMEM((2, _C, _PD), jnp.float32),
            pltpu.VMEM((_NBUF, _NIPC, _PD), jnp.float32),
            pltpu.SemaphoreType.DMA,
            pltpu.SemaphoreType.DMA,
        ],
    )

_CBK = 8192


def _cat_body(u_ref, v_ref, o_ref):
    o_ref[0:_D, :] = u_ref[...]
    o_ref[_D:_PD, :] = v_ref[...]


_cat_call = pl.pallas_call(
    _cat_body,
    grid=((_V + _CBK - 1) // _CBK,),
    in_specs=[
        pl.BlockSpec((_D, _CBK), lambda i: (0, i)),
        pl.BlockSpec((_D, _CBK), lambda i: (0, i)),
    ],
    out_specs=pl.BlockSpec((_PD, _CBK), lambda i: (0, i)),
    out_shape=jax.ShapeDtypeStruct((_PD, _V), jnp.float32),
)

_VBK = 512
_LBK = 2048


def _vis_body(vp_ref, w_ref, b_ref, out_ref):
    y = lax.dot_general(vp_ref[...], w_ref[...], (((1,), (1,)), ((), ())),
                        preferred_element_type=jnp.float32) + b_ref[...]
    n = jnp.sqrt(jnp.sum(y * y, axis=1, keepdims=True))
    out_ref[...] = y / n


_vis_call = pl.pallas_call(
    _vis_body,
    grid=(_B // _VBK,),
    in_specs=[
        pl.BlockSpec((_VBK, _IMG), lambda i: (i, 0)),
        pl.BlockSpec((_D, _IMG), lambda i: (0, 0)),
        pl.BlockSpec((1, _D), lambda i: (0, 0)),
    ],
    out_specs=pl.BlockSpec((_VBK, _D), lambda i: (i, 0)),
    out_shape=jax.ShapeDtypeStruct((_B, _D), jnp.float32),
)


def _logsig(x):
    return jnp.minimum(x, 0.0) - jnp.log(1.0 + jnp.exp(-jnp.abs(x)))


def _loss_body(eu_ref, ev_ref, evs_ref, vis_ref, gw_ref, gb_ref, out_ref):
    i = pl.program_id(0)
    eu = eu_ref[...]
    z = lax.dot_general(eu, gw_ref[...], (((1,), (1,)), ((), ())),
                        preferred_element_type=jnp.float32) + gb_ref[...]
    gate = 1.0 / (1.0 + jnp.exp(-z))
    joint = eu + gate * vis_ref[...]
    score = jnp.sum(joint * ev_ref[...], axis=1)
    negsum = jnp.sum(joint * evs_ref[...], axis=1)
    part = jnp.sum(_logsig(score) + _logsig(-negsum))

    @pl.when(i == 0)
    def _():
        out_ref[...] = jnp.zeros((1, 1), jnp.float32)

    out_ref[...] += part


_loss_call = pl.pallas_call(
    _loss_body,
    grid=(_B // _LBK,),
    in_specs=[
        pl.BlockSpec((_LBK, _D), lambda i: (i, 0)),
        pl.BlockSpec((_LBK, _D), lambda i: (i, 0)),
        pl.BlockSpec((_LBK, _D), lambda i: (i, 0)),
        pl.BlockSpec((_LBK, _D), lambda i: (i, 0)),
        pl.BlockSpec((_D, _D), lambda i: (0, 0)),
        pl.BlockSpec((1, _D), lambda i: (0, 0)),
    ],
    out_specs=pl.BlockSpec((1, 1), lambda i: (0, 0)),
    out_shape=jax.ShapeDtypeStruct((1, 1), jnp.float32),
)


def kernel(u_pos, v_pos, v_neg, visual_pos, batch_size,
           U_emb, V_emb, gate_W, gate_b, img_W, img_b):
    u = u_pos.astype(jnp.int32)
    v = v_pos.astype(jnp.int32)
    vn = v_neg.astype(jnp.int32).reshape(-1)
    Wt = _cat_call(U_emb.T, V_emb.T)
    T = Wt.T
    eu, ev, evs = _make_sc_gather()(u, v, vn, T)
    vis = _vis_call(visual_pos, img_W, img_b.reshape(1, _D))
    res = _loss_call(eu, ev, evs, vis, gate_W, gate_b.reshape(1, _D))
    return -res[0, 0] / batch_size


# speedup vs baseline: 1.9682x; 1.4790x over previous
import functools

import jax
import jax.numpy as jnp
from jax import lax
from jax.experimental import pallas as pl
from jax.experimental.pallas import tpu as pltpu
from jax.experimental.pallas import tpu_sc as plsc

_V = 1000000
_D = 64
_IMG = 2048
_B = 16384
_NNEG = 20

_NW = 32
_BPW = _B // _NW
_C = 128
_NSB = _BPW // _C


_NR = 4
_NIPC = _NR * _NNEG
_NCH = _BPW // _NR
_NBUF = 2


_PD = 2 * _D


def _sc_gather_body(u_hbm, v_hbm, vneg_hbm, T_hbm,
                    eu_out, ev_out, evs_out,
                    uidx, vidx, nidx, acc, pbuf, nbuf,
                    sem_uv, sem_n):
    wid = lax.axis_index("s") * 2 + lax.axis_index("c")
    base = wid * _BPW
    pltpu.sync_copy(u_hbm.at[pl.ds(base, _BPW)], uidx)
    pltpu.sync_copy(v_hbm.at[pl.ds(base, _BPW)], vidx)
    pltpu.sync_copy(vneg_hbm.at[pl.ds(base * _NNEG, _BPW * _NNEG)], nidx)

    jobs = []
    for k in range(_NSB):
        jobs.append((uidx, 0, eu_out, k))
        jobs.append((vidx, _D, ev_out, k))

    def fire_uv(t):
        idx, _col, _out, k = jobs[t]
        return pltpu.async_copy(
            T_hbm.at[idx.at[pl.ds(k * _C, _C)]], pbuf.at[t % 2], sem_uv)

    cps = [fire_uv(0), fire_uv(1)]
    for t in range(2 * _NSB):
        cps[t].wait()
        _idx, col, out, k = jobs[t]
        if t + 2 < 2 * _NSB:
            cps.append(fire_uv(t + 2))
        pltpu.sync_copy(pbuf.at[t % 2, :, pl.ds(col, _D)],
                        out.at[pl.ds(base + k * _C, _C)])

    def fire_n(c, buf):
        return pltpu.async_copy(
            T_hbm.at[nidx.at[pl.ds(c * _NIPC, _NIPC)]], nbuf.at[buf], sem_n)

    for b in range(_NBUF):
        fire_n(b, b)

    def nbody(i, carry):
        for par in range(_NBUF):
            c = i * _NBUF + par
            pltpu.make_async_copy(
                T_hbm.at[nidx.at[pl.ds(0, _NIPC)]], nbuf.at[par], sem_n).wait()
            for rr in range(_NR):
                for c4 in range(_D // 16):
                    sl = pl.ds(_D + c4 * 16, 16)
                    x = nbuf[par, rr * _NNEG, sl]
                    for n in range(1, _NNEG):
                        x = x + nbuf[par, rr * _NNEG + n, sl]
                    acc[c * _NR + rr, pl.ds(c4 * 16, 16)] = x

            @pl.when(c + _NBUF < _NCH)
            def _():
                fire_n(c + _NBUF, par)
        return carry

    lax.fori_loop(0, _NCH // _NBUF, nbody, 0)

    pltpu.sync_copy(acc, evs_out.at[pl.ds(base, _BPW)])


@functools.cache
def _make_sc_gather():
    return pl.kernel(
        _sc_gather_body,
        out_type=(
            jax.ShapeDtypeStruct((_B, _D), jnp.float32),
            jax.ShapeDtypeStruct((_B, _D), jnp.float32),
            jax.ShapeDtypeStruct((_B, _D), jnp.float32),
        ),
        mesh=plsc.VectorSubcoreMesh(core_axis_name="c", subcore_axis_name="s"),
        compiler_params=pltpu.CompilerParams(use_tc_tiling_on_sc=False,
                                             needs_layout_passes=False),
        scratch_types=[
            pltpu.VMEM((_BPW,), jnp.int32),
            pltpu.VMEM((_BPW,), jnp.int32),
            pltpu.VMEM((_BPW * _NNEG,), jnp.int32),
            pltpu.VMEM((_BPW, _D), jnp.float32),
            pltpu.VMEM((2, _C, _PD), jnp.float32),
            pltpu.VMEM((_NBUF, _NIPC, _PD), jnp.float32),
            pltpu.SemaphoreType.DMA,
            pltpu.SemaphoreType.DMA,
        ],
    )

_CBK = 8192


def _cat_body(u_ref, v_ref, o_ref):
    o_ref[:, 0:_D] = u_ref[...].T
    o_ref[:, _D:_PD] = v_ref[...].T


_cat_call = pl.pallas_call(
    _cat_body,
    grid=((_V + _CBK - 1) // _CBK,),
    in_specs=[
        pl.BlockSpec((_D, _CBK), lambda i: (0, i)),
        pl.BlockSpec((_D, _CBK), lambda i: (0, i)),
    ],
    out_specs=pl.BlockSpec((_CBK, _PD), lambda i: (i, 0)),
    out_shape=jax.ShapeDtypeStruct((_V, _PD), jnp.float32),
)

_VBK = 512
_LBK = 2048


def _vis_body(vp_ref, w_ref, b_ref, out_ref):
    y = lax.dot_general(vp_ref[...], w_ref[...], (((1,), (1,)), ((), ())),
                        preferred_element_type=jnp.float32) + b_ref[...]
    n = jnp.sqrt(jnp.sum(y * y, axis=1, keepdims=True))
    out_ref[...] = y / n


_vis_call = pl.pallas_call(
    _vis_body,
    grid=(_B // _VBK,),
    in_specs=[
        pl.BlockSpec((_VBK, _IMG), lambda i: (i, 0)),
        pl.BlockSpec((_D, _IMG), lambda i: (0, 0)),
        pl.BlockSpec((1, _D), lambda i: (0, 0)),
    ],
    out_specs=pl.BlockSpec((_VBK, _D), lambda i: (i, 0)),
    out_shape=jax.ShapeDtypeStruct((_B, _D), jnp.float32),
)


def _logsig(x):
    return jnp.minimum(x, 0.0) - jnp.log(1.0 + jnp.exp(-jnp.abs(x)))


def _loss_body(eu_ref, ev_ref, evs_ref, vis_ref, gw_ref, gb_ref, out_ref):
    i = pl.program_id(0)
    eu = eu_ref[...]
    z = lax.dot_general(eu, gw_ref[...], (((1,), (1,)), ((), ())),
                        preferred_element_type=jnp.float32) + gb_ref[...]
    gate = 1.0 / (1.0 + jnp.exp(-z))
    joint = eu + gate * vis_ref[...]
    score = jnp.sum(joint * ev_ref[...], axis=1)
    negsum = jnp.sum(joint * evs_ref[...], axis=1)
    part = jnp.sum(_logsig(score) + _logsig(-negsum))

    @pl.when(i == 0)
    def _():
        out_ref[...] = jnp.zeros((1, 1), jnp.float32)

    out_ref[...] += part


_loss_call = pl.pallas_call(
    _loss_body,
    grid=(_B // _LBK,),
    in_specs=[
        pl.BlockSpec((_LBK, _D), lambda i: (i, 0)),
        pl.BlockSpec((_LBK, _D), lambda i: (i, 0)),
        pl.BlockSpec((_LBK, _D), lambda i: (i, 0)),
        pl.BlockSpec((_LBK, _D), lambda i: (i, 0)),
        pl.BlockSpec((_D, _D), lambda i: (0, 0)),
        pl.BlockSpec((1, _D), lambda i: (0, 0)),
    ],
    out_specs=pl.BlockSpec((1, 1), lambda i: (0, 0)),
    out_shape=jax.ShapeDtypeStruct((1, 1), jnp.float32),
)


def kernel(u_pos, v_pos, v_neg, visual_pos, batch_size,
           U_emb, V_emb, gate_W, gate_b, img_W, img_b):
    u = u_pos.astype(jnp.int32)
    v = v_pos.astype(jnp.int32)
    vn = v_neg.astype(jnp.int32).reshape(-1)
    T = _cat_call(U_emb.T, V_emb.T)
    eu, ev, evs = _make_sc_gather()(u, v, vn, T)
    vis = _vis_call(visual_pos, img_W, img_b.reshape(1, _D))
    res = _loss_call(eu, ev, evs, vis, gate_W, gate_b.reshape(1, _D))
    return -res[0, 0] / batch_size


# speedup vs baseline: 2.0042x; 1.0183x over previous
import functools

import jax
import jax.numpy as jnp
from jax import lax
from jax.experimental import pallas as pl
from jax.experimental.pallas import tpu as pltpu
from jax.experimental.pallas import tpu_sc as plsc

_V = 1000000
_D = 64
_IMG = 2048
_B = 16384
_NNEG = 20

_NW = 32
_BPW = _B // _NW
_C = 128
_NSB = _BPW // _C


_NR = 4
_NIPC = _NR * _NNEG
_NCH = _BPW // _NR
_NBUF = 4


_PD = 2 * _D


def _sc_gather_body(u_hbm, v_hbm, vneg_hbm, T_hbm,
                    eu_out, ev_out, evs_out,
                    uidx, vidx, nidx, acc, pbuf, nbuf,
                    sem_uv, sem_n):
    wid = lax.axis_index("s") * 2 + lax.axis_index("c")
    base = wid * _BPW
    pltpu.sync_copy(u_hbm.at[pl.ds(base, _BPW)], uidx)
    pltpu.sync_copy(v_hbm.at[pl.ds(base, _BPW)], vidx)
    pltpu.sync_copy(vneg_hbm.at[pl.ds(base * _NNEG, _BPW * _NNEG)], nidx)

    jobs = []
    for k in range(_NSB):
        jobs.append((uidx, 0, eu_out, k))
        jobs.append((vidx, _D, ev_out, k))

    def fire_uv(t):
        idx, _col, _out, k = jobs[t]
        return pltpu.async_copy(
            T_hbm.at[idx.at[pl.ds(k * _C, _C)]], pbuf.at[t % 2], sem_uv)

    cps = [fire_uv(0), fire_uv(1)]
    for t in range(2 * _NSB):
        cps[t].wait()
        _idx, col, out, k = jobs[t]
        if t + 2 < 2 * _NSB:
            cps.append(fire_uv(t + 2))
        pltpu.sync_copy(pbuf.at[t % 2, :, pl.ds(col, _D)],
                        out.at[pl.ds(base + k * _C, _C)])

    def fire_n(c, buf):
        return pltpu.async_copy(
            T_hbm.at[nidx.at[pl.ds(c * _NIPC, _NIPC)]], nbuf.at[buf], sem_n)

    for b in range(_NBUF):
        fire_n(b, b)

    def nbody(i, carry):
        for par in range(_NBUF):
            c = i * _NBUF + par
            pltpu.make_async_copy(
                T_hbm.at[nidx.at[pl.ds(0, _NIPC)]], nbuf.at[par], sem_n).wait()
            for rr in range(_NR):
                for c4 in range(_D // 16):
                    sl = pl.ds(_D + c4 * 16, 16)
                    vals = [nbuf[par, rr * _NNEG + n, sl]
                            for n in range(_NNEG)]
                    while len(vals) > 1:
                        vals = [a + b for a, b in zip(vals[::2], vals[1::2])] \
                            + ([vals[-1]] if len(vals) % 2 else [])
                    acc[c * _NR + rr, pl.ds(c4 * 16, 16)] = vals[0]

            @pl.when(c + _NBUF < _NCH)
            def _():
                fire_n(c + _NBUF, par)
        return carry

    lax.fori_loop(0, _NCH // _NBUF, nbody, 0)

    pltpu.sync_copy(acc, evs_out.at[pl.ds(base, _BPW)])


@functools.cache
def _make_sc_gather():
    return pl.kernel(
        _sc_gather_body,
        out_type=(
            jax.ShapeDtypeStruct((_B, _D), jnp.float32),
            jax.ShapeDtypeStruct((_B, _D), jnp.float32),
            jax.ShapeDtypeStruct((_B, _D), jnp.float32),
        ),
        mesh=plsc.VectorSubcoreMesh(core_axis_name="c", subcore_axis_name="s"),
        compiler_params=pltpu.CompilerParams(use_tc_tiling_on_sc=False,
                                             needs_layout_passes=False),
        scratch_types=[
            pltpu.VMEM((_BPW,), jnp.int32),
            pltpu.VMEM((_BPW,), jnp.int32),
            pltpu.VMEM((_BPW * _NNEG,), jnp.int32),
            pltpu.VMEM((_BPW, _D), jnp.float32),
            pltpu.VMEM((2, _C, _PD), jnp.float32),
            pltpu.VMEM((_NBUF, _NIPC, _PD), jnp.float32),
            pltpu.SemaphoreType.DMA,
            pltpu.SemaphoreType.DMA,
        ],
    )

_CBK = 8192


def _cat_body(u_ref, v_ref, o_ref):
    o_ref[:, 0:_D] = u_ref[...].T
    o_ref[:, _D:_PD] = v_ref[...].T


_cat_call = pl.pallas_call(
    _cat_body,
    grid=((_V + _CBK - 1) // _CBK,),
    in_specs=[
        pl.BlockSpec((_D, _CBK), lambda i: (0, i)),
        pl.BlockSpec((_D, _CBK), lambda i: (0, i)),
    ],
    out_specs=pl.BlockSpec((_CBK, _PD), lambda i: (i, 0)),
    out_shape=jax.ShapeDtypeStruct((_V, _PD), jnp.float32),
)

_VBK = 512
_LBK = 2048


def _vis_body(vp_ref, w_ref, b_ref, out_ref):
    y = lax.dot_general(vp_ref[...], w_ref[...], (((1,), (1,)), ((), ())),
                        preferred_element_type=jnp.float32) + b_ref[...]
    n = jnp.sqrt(jnp.sum(y * y, axis=1, keepdims=True))
    out_ref[...] = y / n


_vis_call = pl.pallas_call(
    _vis_body,
    grid=(_B // _VBK,),
    in_specs=[
        pl.BlockSpec((_VBK, _IMG), lambda i: (i, 0)),
        pl.BlockSpec((_D, _IMG), lambda i: (0, 0)),
        pl.BlockSpec((1, _D), lambda i: (0, 0)),
    ],
    out_specs=pl.BlockSpec((_VBK, _D), lambda i: (i, 0)),
    out_shape=jax.ShapeDtypeStruct((_B, _D), jnp.float32),
)


def _logsig(x):
    return jnp.minimum(x, 0.0) - jnp.log(1.0 + jnp.exp(-jnp.abs(x)))


def _loss_body(eu_ref, ev_ref, evs_ref, vis_ref, gw_ref, gb_ref, out_ref):
    i = pl.program_id(0)
    eu = eu_ref[...]
    z = lax.dot_general(eu, gw_ref[...], (((1,), (1,)), ((), ())),
                        preferred_element_type=jnp.float32) + gb_ref[...]
    gate = 1.0 / (1.0 + jnp.exp(-z))
    joint = eu + gate * vis_ref[...]
    score = jnp.sum(joint * ev_ref[...], axis=1)
    negsum = jnp.sum(joint * evs_ref[...], axis=1)
    part = jnp.sum(_logsig(score) + _logsig(-negsum))

    @pl.when(i == 0)
    def _():
        out_ref[...] = jnp.zeros((1, 1), jnp.float32)

    out_ref[...] += part


_loss_call = pl.pallas_call(
    _loss_body,
    grid=(_B // _LBK,),
    in_specs=[
        pl.BlockSpec((_LBK, _D), lambda i: (i, 0)),
        pl.BlockSpec((_LBK, _D), lambda i: (i, 0)),
        pl.BlockSpec((_LBK, _D), lambda i: (i, 0)),
        pl.BlockSpec((_LBK, _D), lambda i: (i, 0)),
        pl.BlockSpec((_D, _D), lambda i: (0, 0)),
        pl.BlockSpec((1, _D), lambda i: (0, 0)),
    ],
    out_specs=pl.BlockSpec((1, 1), lambda i: (0, 0)),
    out_shape=jax.ShapeDtypeStruct((1, 1), jnp.float32),
)


def kernel(u_pos, v_pos, v_neg, visual_pos, batch_size,
           U_emb, V_emb, gate_W, gate_b, img_W, img_b):
    u = u_pos.astype(jnp.int32)
    v = v_pos.astype(jnp.int32)
    vn = v_neg.astype(jnp.int32).reshape(-1)
    T = _cat_call(U_emb.T, V_emb.T)
    eu, ev, evs = _make_sc_gather()(u, v, vn, T)
    vis = _vis_call(visual_pos, img_W, img_b.reshape(1, _D))
    res = _loss_call(eu, ev, evs, vis, gate_W, gate_b.reshape(1, _D))
    return -res[0, 0] / batch_size


# speedup vs baseline: 2.1169x; 1.0562x over previous
import functools

import jax
import jax.numpy as jnp
from jax import lax
from jax.experimental import pallas as pl
from jax.experimental.pallas import tpu as pltpu
from jax.experimental.pallas import tpu_sc as plsc

_V = 1000000
_D = 64
_IMG = 2048
_B = 16384
_NNEG = 20

_NW = 32
_BPW = _B // _NW
_C = 128
_NSB = _BPW // _C


_NR = 4
_NIPC = _NR * _NNEG
_NCH = _BPW // _NR
_NBUF = 4


_PD = 2 * _D


def _sc_gather_body(u_hbm, v_hbm, vneg_hbm, T_hbm,
                    eu_out, ev_out, evs_out,
                    uidx, vidx, nidx, acc, pbuf, nbuf,
                    sem_uv, sem_n):
    wid = lax.axis_index("s") * 2 + lax.axis_index("c")
    base = wid * _BPW
    pltpu.sync_copy(u_hbm.at[pl.ds(base, _BPW)], uidx)
    pltpu.sync_copy(v_hbm.at[pl.ds(base, _BPW)], vidx)
    pltpu.sync_copy(vneg_hbm.at[pl.ds(base * _NNEG, _BPW * _NNEG)], nidx)

    jobs = []
    for k in range(_NSB):
        jobs.append((uidx, 0, eu_out, k))
        jobs.append((vidx, _D, ev_out, k))

    def fire_uv(t):
        idx, _col, _out, k = jobs[t]
        return pltpu.async_copy(
            T_hbm.at[idx.at[pl.ds(k * _C, _C)]], pbuf.at[t % 2], sem_uv)

    cps = [fire_uv(0), fire_uv(1)]
    for t in range(2 * _NSB):
        cps[t].wait()
        _idx, col, out, k = jobs[t]
        if t + 2 < 2 * _NSB:
            cps.append(fire_uv(t + 2))
        pltpu.sync_copy(pbuf.at[t % 2, :, pl.ds(col, _D)],
                        out.at[pl.ds(base + k * _C, _C)])

    def fire_n(c, buf):
        return pltpu.async_copy(
            T_hbm.at[nidx.at[pl.ds(c * _NIPC, _NIPC)]], nbuf.at[buf], sem_n)

    for b in range(_NBUF):
        fire_n(b, b)

    def nbody(i, carry):
        for par in range(_NBUF):
            c = i * _NBUF + par
            pltpu.make_async_copy(
                T_hbm.at[nidx.at[pl.ds(0, _NIPC)]], nbuf.at[par], sem_n).wait()
            for rr in range(_NR):
                for c4 in range(_D // 16):
                    sl = pl.ds(_D + c4 * 16, 16)
                    vals = [nbuf[par, rr * _NNEG + n, sl]
                            for n in range(_NNEG)]
                    while len(vals) > 1:
                        vals = [a + b for a, b in zip(vals[::2], vals[1::2])] \
                            + ([vals[-1]] if len(vals) % 2 else [])
                    acc[c * _NR + rr, pl.ds(c4 * 16, 16)] = vals[0]

            @pl.when(c + _NBUF < _NCH)
            def _():
                fire_n(c + _NBUF, par)
        return carry

    lax.fori_loop(0, _NCH // _NBUF, nbody, 0)

    pltpu.sync_copy(acc, evs_out.at[pl.ds(base, _BPW)])


@functools.cache
def _make_sc_gather():
    return pl.kernel(
        _sc_gather_body,
        out_type=(
            jax.ShapeDtypeStruct((_B, _D), jnp.float32),
            jax.ShapeDtypeStruct((_B, _D), jnp.float32),
            jax.ShapeDtypeStruct((_B, _D), jnp.float32),
        ),
        mesh=plsc.VectorSubcoreMesh(core_axis_name="c", subcore_axis_name="s"),
        compiler_params=pltpu.CompilerParams(use_tc_tiling_on_sc=False,
                                             needs_layout_passes=False),
        scratch_types=[
            pltpu.VMEM((_BPW,), jnp.int32),
            pltpu.VMEM((_BPW,), jnp.int32),
            pltpu.VMEM((_BPW * _NNEG,), jnp.int32),
            pltpu.VMEM((_BPW, _D), jnp.float32),
            pltpu.VMEM((2, _C, _PD), jnp.float32),
            pltpu.VMEM((_NBUF, _NIPC, _PD), jnp.float32),
            pltpu.SemaphoreType.DMA,
            pltpu.SemaphoreType.DMA,
        ],
    )

_CBK = 16384


def _cat_body(u_ref, v_ref, o_ref):
    o_ref[:, 0:_D] = u_ref[...].T
    o_ref[:, _D:_PD] = v_ref[...].T


_cat_call = pl.pallas_call(
    _cat_body,
    grid=((_V + _CBK - 1) // _CBK,),
    in_specs=[
        pl.BlockSpec((_D, _CBK), lambda i: (0, i)),
        pl.BlockSpec((_D, _CBK), lambda i: (0, i)),
    ],
    out_specs=pl.BlockSpec((_CBK, _PD), lambda i: (i, 0)),
    out_shape=jax.ShapeDtypeStruct((_V, _PD), jnp.float32),
)

_VBK = 512
_LBK = 2048


def _vis_body(vp_ref, w_ref, b_ref, out_ref):
    y = lax.dot_general(vp_ref[...], w_ref[...], (((1,), (1,)), ((), ())),
                        preferred_element_type=jnp.float32) + b_ref[...]
    n = jnp.sqrt(jnp.sum(y * y, axis=1, keepdims=True))
    out_ref[...] = y / n


_vis_call = pl.pallas_call(
    _vis_body,
    grid=(_B // _VBK,),
    in_specs=[
        pl.BlockSpec((_VBK, _IMG), lambda i: (i, 0)),
        pl.BlockSpec((_D, _IMG), lambda i: (0, 0)),
        pl.BlockSpec((1, _D), lambda i: (0, 0)),
    ],
    out_specs=pl.BlockSpec((_VBK, _D), lambda i: (i, 0)),
    out_shape=jax.ShapeDtypeStruct((_B, _D), jnp.float32),
)


def _logsig(x):
    return jnp.minimum(x, 0.0) - jnp.log(1.0 + jnp.exp(-jnp.abs(x)))


def _loss_body(eu_ref, ev_ref, evs_ref, vis_ref, gw_ref, gb_ref, out_ref):
    i = pl.program_id(0)
    eu = eu_ref[...]
    z = lax.dot_general(eu, gw_ref[...], (((1,), (1,)), ((), ())),
                        preferred_element_type=jnp.float32) + gb_ref[...]
    gate = 1.0 / (1.0 + jnp.exp(-z))
    joint = eu + gate * vis_ref[...]
    score = jnp.sum(joint * ev_ref[...], axis=1)
    negsum = jnp.sum(joint * evs_ref[...], axis=1)
    part = jnp.sum(_logsig(score) + _logsig(-negsum))

    @pl.when(i == 0)
    def _():
        out_ref[...] = jnp.zeros((1, 1), jnp.float32)

    out_ref[...] += part


_loss_call = pl.pallas_call(
    _loss_body,
    grid=(_B // _LBK,),
    in_specs=[
        pl.BlockSpec((_LBK, _D), lambda i: (i, 0)),
        pl.BlockSpec((_LBK, _D), lambda i: (i, 0)),
        pl.BlockSpec((_LBK, _D), lambda i: (i, 0)),
        pl.BlockSpec((_LBK, _D), lambda i: (i, 0)),
        pl.BlockSpec((_D, _D), lambda i: (0, 0)),
        pl.BlockSpec((1, _D), lambda i: (0, 0)),
    ],
    out_specs=pl.BlockSpec((1, 1), lambda i: (0, 0)),
    out_shape=jax.ShapeDtypeStruct((1, 1), jnp.float32),
)


def kernel(u_pos, v_pos, v_neg, visual_pos, batch_size,
           U_emb, V_emb, gate_W, gate_b, img_W, img_b):
    u = u_pos.astype(jnp.int32)
    v = v_pos.astype(jnp.int32)
    vn = v_neg.astype(jnp.int32).reshape(-1)
    T = _cat_call(U_emb.T, V_emb.T)
    eu, ev, evs = _make_sc_gather()(u, v, vn, T)
    vis = _vis_call(visual_pos, img_W, img_b.reshape(1, _D))
    res = _loss_call(eu, ev, evs, vis, gate_W, gate_b.reshape(1, _D))
    return -res[0, 0] / batch_size


# speedup vs baseline: 2.1261x; 1.0044x over previous
import functools

import jax
import jax.numpy as jnp
from jax import lax
from jax.experimental import pallas as pl
from jax.experimental.pallas import tpu as pltpu
from jax.experimental.pallas import tpu_sc as plsc

_V = 1000000
_D = 64
_IMG = 2048
_B = 16384
_NNEG = 20

_NW = 32
_BPW = _B // _NW
_C = 128
_NSB = _BPW // _C


_NR = 4
_NIPC = _NR * _NNEG
_NCH = _BPW // _NR
_NBUF = 4


_PD = 2 * _D


def _sc_gather_body(u_hbm, v_hbm, vneg_hbm, T_hbm,
                    eu_out, ev_out, evs_out,
                    uidx, vidx, nidx, acc, pbuf, nbuf,
                    sem_uv, sem_n):
    wid = lax.axis_index("s") * 2 + lax.axis_index("c")
    base = wid * _BPW
    pltpu.sync_copy(u_hbm.at[pl.ds(base, _BPW)], uidx)
    pltpu.sync_copy(v_hbm.at[pl.ds(base, _BPW)], vidx)
    pltpu.sync_copy(vneg_hbm.at[pl.ds(base * _NNEG, _BPW * _NNEG)], nidx)

    jobs = []
    for k in range(_NSB):
        jobs.append((uidx, 0, eu_out, k))
        jobs.append((vidx, _D, ev_out, k))

    def fire_uv(t):
        idx, _col, _out, k = jobs[t]
        return pltpu.async_copy(
            T_hbm.at[idx.at[pl.ds(k * _C, _C)]], pbuf.at[t % 2], sem_uv)

    cps = [fire_uv(0), fire_uv(1)]
    for t in range(2 * _NSB):
        cps[t].wait()
        _idx, col, out, k = jobs[t]
        if t + 2 < 2 * _NSB:
            cps.append(fire_uv(t + 2))
        pltpu.sync_copy(pbuf.at[t % 2, :, pl.ds(col, _D)],
                        out.at[pl.ds(base + k * _C, _C)])

    def fire_n(c, buf):
        return pltpu.async_copy(
            T_hbm.at[nidx.at[pl.ds(c * _NIPC, _NIPC)]], nbuf.at[buf], sem_n)

    for b in range(_NBUF):
        fire_n(b, b)

    def nbody(i, carry):
        for par in range(_NBUF):
            c = i * _NBUF + par
            pltpu.make_async_copy(
                T_hbm.at[nidx.at[pl.ds(0, _NIPC)]], nbuf.at[par], sem_n).wait()
            for rr in range(_NR):
                for c4 in range(_D // 16):
                    sl = pl.ds(_D + c4 * 16, 16)
                    vals = [nbuf[par, rr * _NNEG + n, sl]
                            for n in range(_NNEG)]
                    while len(vals) > 1:
                        vals = [a + b for a, b in zip(vals[::2], vals[1::2])] \
                            + ([vals[-1]] if len(vals) % 2 else [])
                    acc[c * _NR + rr, pl.ds(c4 * 16, 16)] = vals[0]

            @pl.when(c + _NBUF < _NCH)
            def _():
                fire_n(c + _NBUF, par)
        return carry

    lax.fori_loop(0, _NCH // _NBUF, nbody, 0)

    pltpu.sync_copy(acc, evs_out.at[pl.ds(base, _BPW)])


@functools.cache
def _make_sc_gather():
    return pl.kernel(
        _sc_gather_body,
        out_type=(
            jax.ShapeDtypeStruct((_B, _D), jnp.float32),
            jax.ShapeDtypeStruct((_B, _D), jnp.float32),
            jax.ShapeDtypeStruct((_B, _D), jnp.float32),
        ),
        mesh=plsc.VectorSubcoreMesh(core_axis_name="c", subcore_axis_name="s"),
        compiler_params=pltpu.CompilerParams(use_tc_tiling_on_sc=False,
                                             needs_layout_passes=False),
        scratch_types=[
            pltpu.VMEM((_BPW,), jnp.int32),
            pltpu.VMEM((_BPW,), jnp.int32),
            pltpu.VMEM((_BPW * _NNEG,), jnp.int32),
            pltpu.VMEM((_BPW, _D), jnp.float32),
            pltpu.VMEM((2, _C, _PD), jnp.float32),
            pltpu.VMEM((_NBUF, _NIPC, _PD), jnp.float32),
            pltpu.SemaphoreType.DMA,
            pltpu.SemaphoreType.DMA,
        ],
    )

_CBK = 20480


def _cat_body(u_ref, v_ref, o_ref):
    o_ref[:, 0:_D] = u_ref[...].T
    o_ref[:, _D:_PD] = v_ref[...].T


_cat_call = pl.pallas_call(
    _cat_body,
    grid=((_V + _CBK - 1) // _CBK,),
    in_specs=[
        pl.BlockSpec((_D, _CBK), lambda i: (0, i)),
        pl.BlockSpec((_D, _CBK), lambda i: (0, i)),
    ],
    out_specs=pl.BlockSpec((_CBK, _PD), lambda i: (i, 0)),
    out_shape=jax.ShapeDtypeStruct((_V, _PD), jnp.float32),
)

_VBK = 512
_LBK = 2048


def _vis_body(vp_ref, w_ref, b_ref, out_ref):
    y = lax.dot_general(vp_ref[...], w_ref[...], (((1,), (1,)), ((), ())),
                        preferred_element_type=jnp.float32) + b_ref[...]
    n = jnp.sqrt(jnp.sum(y * y, axis=1, keepdims=True))
    out_ref[...] = y / n


_vis_call = pl.pallas_call(
    _vis_body,
    grid=(_B // _VBK,),
    in_specs=[
        pl.BlockSpec((_VBK, _IMG), lambda i: (i, 0)),
        pl.BlockSpec((_D, _IMG), lambda i: (0, 0)),
        pl.BlockSpec((1, _D), lambda i: (0, 0)),
    ],
    out_specs=pl.BlockSpec((_VBK, _D), lambda i: (i, 0)),
    out_shape=jax.ShapeDtypeStruct((_B, _D), jnp.float32),
)


def _logsig(x):
    return jnp.minimum(x, 0.0) - jnp.log(1.0 + jnp.exp(-jnp.abs(x)))


def _loss_body(eu_ref, ev_ref, evs_ref, vis_ref, gw_ref, gb_ref, out_ref):
    i = pl.program_id(0)
    eu = eu_ref[...]
    z = lax.dot_general(eu, gw_ref[...], (((1,), (1,)), ((), ())),
                        preferred_element_type=jnp.float32) + gb_ref[...]
    gate = 1.0 / (1.0 + jnp.exp(-z))
    joint = eu + gate * vis_ref[...]
    score = jnp.sum(joint * ev_ref[...], axis=1)
    negsum = jnp.sum(joint * evs_ref[...], axis=1)
    part = jnp.sum(_logsig(score) + _logsig(-negsum))

    @pl.when(i == 0)
    def _():
        out_ref[...] = jnp.zeros((1, 1), jnp.float32)

    out_ref[...] += part


_loss_call = pl.pallas_call(
    _loss_body,
    grid=(_B // _LBK,),
    in_specs=[
        pl.BlockSpec((_LBK, _D), lambda i: (i, 0)),
        pl.BlockSpec((_LBK, _D), lambda i: (i, 0)),
        pl.BlockSpec((_LBK, _D), lambda i: (i, 0)),
        pl.BlockSpec((_LBK, _D), lambda i: (i, 0)),
        pl.BlockSpec((_D, _D), lambda i: (0, 0)),
        pl.BlockSpec((1, _D), lambda i: (0, 0)),
    ],
    out_specs=pl.BlockSpec((1, 1), lambda i: (0, 0)),
    out_shape=jax.ShapeDtypeStruct((1, 1), jnp.float32),
)


def kernel(u_pos, v_pos, v_neg, visual_pos, batch_size,
           U_emb, V_emb, gate_W, gate_b, img_W, img_b):
    u = u_pos.astype(jnp.int32)
    v = v_pos.astype(jnp.int32)
    vn = v_neg.astype(jnp.int32).reshape(-1)
    T = _cat_call(U_emb.T, V_emb.T)
    eu, ev, evs = _make_sc_gather()(u, v, vn, T)
    vis = _vis_call(visual_pos, img_W, img_b.reshape(1, _D))
    res = _loss_call(eu, ev, evs, vis, gate_W, gate_b.reshape(1, _D))
    return -res[0, 0] / batch_size
